# Initial kernel scaffold; baseline (speedup 1.0000x reference)
import jax
import jax.numpy as jnp
from jax.experimental import pallas as pl


def kernel(x, token_table):
    raise NotImplementedError("write your pallas kernel here")


# speedup vs baseline: 1.0232x; 1.0232x over previous
import functools

import jax
import jax.numpy as jnp
from jax import lax
from jax.experimental import pallas as pl
from jax.experimental.pallas import tpu as pltpu
from jax.experimental.pallas import tpu_sc as plsc

_info = plsc.get_sparse_core_info()
_NC, _NS, _L = _info.num_cores, _info.num_subcores, _info.num_lanes
_NW = _NC * _NS


def _positional_table(seq_length, d_model):
    pos = jnp.arange(seq_length, dtype=jnp.float32)[:, None]
    two_i = jnp.arange(0, d_model, 2, dtype=jnp.float32)
    div = jnp.power(10000.0, two_i / d_model)
    pe = jnp.zeros((seq_length, d_model), dtype=jnp.float32)
    pe = pe.at[:, 0::2].set(jnp.sin(pos / div))
    pe = pe.at[:, 1::2].set(jnp.cos(pos / div))
    return pe


@functools.partial(jax.jit, static_argnums=(3, 4, 5))
def _embed(xf, table, pe, batch, seq, d):
    s_per_w = seq // _NW
    mesh = plsc.VectorSubcoreMesh(core_axis_name="c", subcore_axis_name="s")

    @functools.partial(
        pl.kernel,
        mesh=mesh,
        out_type=jax.ShapeDtypeStruct((batch * seq, d), jnp.float32),
        scratch_types=[
            pltpu.VMEM((s_per_w,), jnp.int32),
            pltpu.VMEM((s_per_w, d), jnp.float32),
            pltpu.VMEM((s_per_w, d), jnp.float32),
            pltpu.SemaphoreType.DMA,
        ],
    )
    def k(x_hbm, table_hbm, pe_hbm, out_hbm, idx_v, pe_v, tok_v, sem):
        wid = lax.axis_index("s") * _NC + lax.axis_index("c")
        s_base = wid * s_per_w
        pltpu.sync_copy(pe_hbm.at[pl.ds(s_base, s_per_w)], pe_v)
        cols = d // _L
        for b in range(batch):
            flat = b * seq + s_base
            pltpu.sync_copy(x_hbm.at[pl.ds(flat, s_per_w)], idx_v)
            pltpu.async_copy(table_hbm.at[idx_v], tok_v, sem).wait()

            def add_row(r, _):
                for c in range(cols):
                    sl = pl.ds(c * _L, _L)
                    tok_v[r, sl] = tok_v[r, sl] + pe_v[r, sl]
                return _

            lax.fori_loop(0, s_per_w, add_row, 0)
            pltpu.sync_copy(tok_v, out_hbm.at[pl.ds(flat, s_per_w)])

    return k(xf, table, pe)


def kernel(x, token_table):
    batch, seq = x.shape
    vocab, d = token_table.shape
    xf = x.reshape(-1).astype(jnp.int32)
    pe = _positional_table(seq, d)
    out = _embed(xf, token_table, pe, batch, seq, d)
    return out.reshape(batch, seq, d)
